# baseline (speedup 1.0000x reference)
import functools

import jax
import jax.numpy as jnp
from jax import lax
from jax.experimental import pallas as pl
from jax.experimental.pallas import tpu as pltpu
from jax.experimental.pallas import tpu_sc as plsc

VOCAB = 1000000
D = 32
B = 16384
F = 26

NC = 2
NS = 16
NW = NC * NS
BW = B // NW

_mesh = plsc.VectorSubcoreMesh(
    core_axis_name="c", subcore_axis_name="s", num_cores=NC, num_subcores=NS
)


@functools.partial(
    pl.kernel,
    out_type=jax.ShapeDtypeStruct((F, D, B), jnp.float32),
    mesh=_mesh,
    scratch_types=[
        pltpu.VMEM((BW,), jnp.int32),
        pltpu.VMEM((BW,), jnp.int32),
        pltpu.VMEM((BW, D), jnp.float32),
        pltpu.VMEM((BW, D), jnp.float32),
        pltpu.VMEM((D, BW), jnp.float32),
        pltpu.VMEM((D, BW), jnp.float32),
        pltpu.SemaphoreType.DMA,
        pltpu.SemaphoreType.DMA,
        pltpu.SemaphoreType.DMA,
        pltpu.SemaphoreType.DMA,
    ],
    compiler_params=pltpu.CompilerParams(
        use_tc_tiling_on_sc=False, needs_layout_passes=False
    ),
)
def _embed_kernel(
    xt_hbm, w_hbm, out_hbm, idx0, idx1, raw0, raw1, tb0, tb1, g0, g1, s0, s1
):
    wid = lax.axis_index("s") * NC + lax.axis_index("c")
    boff = wid * BW

    idxs = (idx0, idx1)
    raws = (raw0, raw1)
    tbs = (tb0, tb1)
    gsems = (g0, g1)
    ssems = (s0, s1)
    lanes = lax.iota(jnp.int32, 16)

    def stage_idx(f, slot):
        pltpu.sync_copy(xt_hbm.at[f, pl.ds(boff, BW)], idxs[slot])

    def fire_gather(slot):
        return pltpu.async_copy(w_hbm.at[idxs[slot]], raws[slot], gsems[slot])

    def wait_gather(slot):
        pltpu.make_async_copy(
            w_hbm.at[idxs[slot]], raws[slot], gsems[slot]
        ).wait()

    def fire_store(f, slot):
        return pltpu.async_copy(
            tbs[slot], out_hbm.at[f, :, pl.ds(boff, BW)], ssems[slot]
        )

    def wait_store(f, slot):
        pltpu.make_async_copy(
            tbs[slot], out_hbm.at[f, :, pl.ds(boff, BW)], ssems[slot]
        ).wait()

    dvecs = [jnp.full((16,), d, jnp.int32) for d in range(D)]

    def transpose(slot):
        raw = raws[slot]
        tb = tbs[slot]

        @pl.loop(0, BW // 16, unroll=2)
        def _grp(t):
            rowv = t * 16 + lanes
            for d in range(D):
                tb[d, pl.ds(t * 16, 16)] = plsc.load_gather(raw, [rowv, dvecs[d]])

    stage_idx(0, 0)
    fire_gather(0)

    @pl.loop(0, F // 2)
    def _pipe(p):
        f0 = 2 * p
        f1 = 2 * p + 1

        wait_gather(0)
        stage_idx(f1, 1)

        @pl.when(p > 0)
        def _():
            wait_store(f1 - 2, 1)

        fire_gather(1)

        @pl.when(p > 0)
        def _():
            wait_store(f0 - 2, 0)

        transpose(0)
        fire_store(f0, 0)

        wait_gather(1)

        @pl.when(p + 1 < F // 2)
        def _():
            stage_idx(f0 + 2, 0)
            fire_gather(0)

        transpose(1)
        fire_store(f1, 1)

    wait_store(F - 2, 0)
    wait_store(F - 1, 1)


def kernel(x, weight):
    xt = jnp.transpose(x).astype(jnp.int32)
    out3 = _embed_kernel(xt, weight)
    return jnp.transpose(out3, (2, 0, 1))


# speedup vs baseline: 1.0650x; 1.0650x over previous
import functools

import jax
import jax.numpy as jnp
from jax import lax
from jax.experimental import pallas as pl
from jax.experimental.pallas import tpu as pltpu
from jax.experimental.pallas import tpu_sc as plsc

VOCAB = 1000000
D = 32
B = 16384
F = 26

NC = 2
NS = 16
NW = NC * NS
BW = B // NW

_mesh = plsc.VectorSubcoreMesh(
    core_axis_name="c", subcore_axis_name="s", num_cores=NC, num_subcores=NS
)


@functools.partial(
    pl.kernel,
    out_type=jax.ShapeDtypeStruct((F * D * B,), jnp.float32),
    mesh=_mesh,
    scratch_types=[
        pltpu.VMEM((BW,), jnp.int32),
        pltpu.VMEM((BW,), jnp.int32),
        pltpu.VMEM((BW, D), jnp.float32),
        pltpu.VMEM((BW, D), jnp.float32),
        pltpu.VMEM((D * BW,), jnp.float32),
        pltpu.VMEM((D * BW,), jnp.float32),
        pltpu.SemaphoreType.DMA,
        pltpu.SemaphoreType.DMA,
        pltpu.SemaphoreType.DMA,
        pltpu.SemaphoreType.DMA,
    ],
    compiler_params=pltpu.CompilerParams(
        use_tc_tiling_on_sc=False, needs_layout_passes=False
    ),
)
def _embed_kernel(
    xt_hbm, w_hbm, out_hbm, idx0, idx1, raw0, raw1, tb0, tb1, g0, g1, s0, s1
):
    wid = lax.axis_index("s") * NC + lax.axis_index("c")
    boff = wid * BW

    idxs = (idx0, idx1)
    raws = (raw0, raw1)
    tbs = (tb0, tb1)
    gsems = (g0, g1)
    ssems = (s0, s1)
    lanes = lax.iota(jnp.int32, 16)
    dst0 = lanes * BW
    dst1 = (lanes + 16) * BW

    def stage_idx(f, slot):
        pltpu.sync_copy(xt_hbm.at[f, pl.ds(boff, BW)], idxs[slot])

    def fire_gather(slot):
        return pltpu.async_copy(w_hbm.at[idxs[slot]], raws[slot], gsems[slot])

    def wait_gather(slot):
        pltpu.make_async_copy(
            w_hbm.at[idxs[slot]], raws[slot], gsems[slot]
        ).wait()

    def fire_store(f, slot):
        for d in range(D):
            pltpu.async_copy(
                tbs[slot].at[pl.ds(d * BW, BW)],
                out_hbm.at[pl.ds((f * D + d) * B + boff, BW)],
                ssems[slot],
            )

    def wait_store(f, slot):
        for d in range(D):
            pltpu.make_async_copy(
                tbs[slot].at[pl.ds(d * BW, BW)],
                out_hbm.at[pl.ds((f * D + d) * B + boff, BW)],
                ssems[slot],
            ).wait()

    def transpose(slot):
        raw = raws[slot]
        tb = tbs[slot]

        @pl.loop(0, BW, unroll=8)
        def _row(r):
            plsc.store_scatter(tb, [dst0 + r], raw[r, pl.ds(0, 16)])
            plsc.store_scatter(tb, [dst1 + r], raw[r, pl.ds(16, 16)])

    stage_idx(0, 0)
    fire_gather(0)

    @pl.loop(0, F // 2)
    def _pipe(p):
        f0 = 2 * p
        f1 = 2 * p + 1

        wait_gather(0)
        stage_idx(f1, 1)
        fire_gather(1)

        @pl.when(p > 0)
        def _():
            wait_store(f0 - 2, 0)

        transpose(0)
        fire_store(f0, 0)

        wait_gather(1)

        @pl.when(p + 1 < F // 2)
        def _():
            stage_idx(f0 + 2, 0)
            fire_gather(0)

        @pl.when(p > 0)
        def _():
            wait_store(f1 - 2, 1)

        transpose(1)
        fire_store(f1, 1)

    wait_store(F - 2, 0)
    wait_store(F - 1, 1)


def kernel(x, weight):
    xt = jnp.transpose(x).astype(jnp.int32)
    out3 = _embed_kernel(xt, weight).reshape(F, D, B)
    return jnp.transpose(out3, (2, 0, 1))


# speedup vs baseline: 1.1449x; 1.0750x over previous
import functools

import jax
import jax.numpy as jnp
from jax import lax
from jax.experimental import pallas as pl
from jax.experimental.pallas import tpu as pltpu
from jax.experimental.pallas import tpu_sc as plsc

VOCAB = 1000000
D = 32
B = 16384
F = 26
N = B * F

NC = 2
NS = 16
NW = NC * NS
ROWS_PER_W = N // NW

N_GROUPS = 8
ROWS_PER_GROUP = ROWS_PER_W // N_GROUPS

_mesh = plsc.VectorSubcoreMesh(
    core_axis_name="c", subcore_axis_name="s", num_cores=NC, num_subcores=NS
)


@functools.partial(
    pl.kernel,
    out_type=jax.ShapeDtypeStruct((N, D), jnp.float32),
    mesh=_mesh,
    scratch_types=[
        pltpu.VMEM((ROWS_PER_W,), jnp.int32),
        pltpu.VMEM((ROWS_PER_GROUP, D), jnp.float32),
        pltpu.VMEM((ROWS_PER_GROUP, D), jnp.float32),
        pltpu.SemaphoreType.DMA,
        pltpu.SemaphoreType.DMA,
        pltpu.SemaphoreType.DMA,
        pltpu.SemaphoreType.DMA,
    ],
    compiler_params=pltpu.CompilerParams(use_tc_tiling_on_sc=False),
)
def _embed_kernel(x_hbm, w_hbm, out_hbm, idx_v, rows0, rows1, g0, g1, s0, s1):
    wid = lax.axis_index("s") * NC + lax.axis_index("c")
    base = wid * ROWS_PER_W
    pltpu.sync_copy(x_hbm.at[pl.ds(base, ROWS_PER_W)], idx_v)

    bufs = (rows0, rows1)
    gsems = (g0, g1)
    ssems = (s0, s1)

    def fire_gather(g, slot):
        return pltpu.async_copy(
            w_hbm.at[idx_v.at[pl.ds(g * ROWS_PER_GROUP, ROWS_PER_GROUP)]],
            bufs[slot],
            gsems[slot],
        )

    def fire_store(g, slot):
        return pltpu.async_copy(
            bufs[slot],
            out_hbm.at[pl.ds(base + g * ROWS_PER_GROUP, ROWS_PER_GROUP)],
            ssems[slot],
        )

    gath = [fire_gather(0, 0), None]
    stor = [None, None]
    for g in range(N_GROUPS):
        s = g & 1
        s2 = s ^ 1
        gath[s].wait()
        if g + 1 < N_GROUPS:
            if stor[s2] is not None:
                stor[s2].wait()
            gath[s2] = fire_gather(g + 1, s2)
        stor[s] = fire_store(g, s)
    stor[0].wait()
    stor[1].wait()


def kernel(x, weight):
    x_flat = x.reshape(-1).astype(jnp.int32)
    out = _embed_kernel(x_flat, weight)
    return out.reshape(B, F, D)
